# baseline (speedup 1.0000x reference)
import functools

import jax
import jax.numpy as jnp
from jax import lax
from jax.experimental import pallas as pl
from jax.experimental.pallas import tpu as pltpu
from jax.experimental.pallas import tpu_sc as plsc

_B = 4096
_L = 50
_D = 32
_NC = 2
_NS = 16
_NW = _NC * _NS
_BPW = _B // _NW


def _sc_text_body(qt_hbm, tt_hbm, text_hbm, qmean_out, tmean_out,
                  tok_idx, b0, b1, b2, b3, b4, b5, b6, b7,
                  mean_v, meant_v, s0, s1, s2, s3, s4, s5, s6, s7):
  wid = lax.axis_index("s") * _NC + lax.axis_index("c")
  base = wid * _BPW
  inv_l = jnp.float32(1.0 / _L)
  zero = jnp.zeros((16,), jnp.float32)
  lane16 = lax.iota(jnp.int32, 16)
  lane32 = lane16 * _D
  bufs = (b0, b1, b2, b3, b4, b5, b6, b7)
  sems = (s0, s1, s2, s3, s4, s5, s6, s7)

  def tower(idxt_hbm, meanout):
    pltpu.sync_copy(idxt_hbm.at[:, pl.ds(base, _BPW)], tok_idx)

    def zinit(g, c):
      for u in range(8):
        mean_v[pl.ds(g * 128 + u * 16, 16)] = zero
      return c

    lax.fori_loop(0, 32, zinit, 0)

    def launch(k, t):
      pltpu.async_copy(text_hbm.at[tok_idx.at[t]], bufs[k], sems[k])

    def wait(k):
      pltpu.make_async_copy(text_hbm.at[tok_idx.at[0]], bufs[k],
                            sems[k]).wait()

    def acc(ks):
      def grp(g, c):
        pend = []
        for u in range(8):
          r = g * 4 + (u // 2)
          o = (u % 2) * 16
          v = bufs[ks[0]][r, pl.ds(o, 16)]
          for k in ks[1:]:
            v = v + bufs[k][r, pl.ds(o, 16)]
          pend.append((g * 128 + u * 16, v))
        for off, v in pend:
          plsc.addupdate(mean_v.at[pl.ds(off, 16)], v)
        return c

      lax.fori_loop(0, 32, grp, 0)

    for k in range(4):
      launch(k, k)

    def body(h, c):
      t0 = 8 * h
      for k in range(4):
        launch(4 + k, jnp.minimum(t0 + 4 + k, _L - 1))
      for k in range(4):
        wait(k)
      acc((0, 1, 2, 3))
      for k in range(4):
        launch(k, jnp.minimum(t0 + 8 + k, _L - 1))
      for k in range(4):
        wait(4 + k)
      acc((4, 5, 6, 7))
      return c

    lax.fori_loop(0, 6, body, 0)
    for k in range(4):
      wait(k)
    acc((0, 1))

    def tpose(f, c):
      fb = jnp.broadcast_to(f, (16,))
      for g in range(8):
        idx = lane32 + (g * 16 * _D + f)
        v = plsc.load_gather(mean_v, [idx])
        plsc.store_scatter(meant_v, [fb, lane16 + g * 16], v * inv_l)
      return c

    lax.fori_loop(0, _D, tpose, 0)
    pltpu.sync_copy(meant_v, meanout.at[:, pl.ds(base, _BPW)])

  tower(qt_hbm, qmean_out)
  tower(tt_hbm, tmean_out)


_sc_text = functools.partial(
    pl.kernel,
    out_type=(
        jax.ShapeDtypeStruct((_D, _B), jnp.float32),
        jax.ShapeDtypeStruct((_D, _B), jnp.float32),
    ),
    mesh=plsc.VectorSubcoreMesh(core_axis_name="c", subcore_axis_name="s"),
    compiler_params=pltpu.CompilerParams(use_tc_tiling_on_sc=False,
                                         needs_layout_passes=False),
    scratch_types=(
        [pltpu.VMEM((_L, _BPW), jnp.int32)]
        + [pltpu.VMEM((_BPW, _D), jnp.float32)] * 8
        + [pltpu.VMEM((_BPW * _D,), jnp.float32),
           pltpu.VMEM((_D, _BPW), jnp.float32)]
        + [pltpu.SemaphoreType.DMA] * 8
    ),
)(_sc_text_body)


def _sc_id_body(uid_hbm, iid_hbm, utabt_hbm, itabt_hbm,
                uout, iout, idx_v, tile_a, tile_b, tile_c, tile_d, cols_v,
                sem_a, sem_b, sem_c, sem_d):
  wid = lax.axis_index("s") * _NC + lax.axis_index("c")
  base = wid * _BPW
  row16 = lax.iota(jnp.int32, 16)

  def tower(id_hbm, tabt_hbm, out):
    pltpu.sync_copy(id_hbm.at[pl.ds(base, _BPW)], idx_v)
    bufs = (tile_a, tile_b, tile_c, tile_d)
    sems = (sem_a, sem_b, sem_c, sem_d)
    all_ids = []
    for g in range(_BPW // 16):
      ids = idx_v[pl.ds(g * 16, 16)]
      all_ids.extend((ids[k], g * 16 + k) for k in range(16))

    def fetch(j, buf, sem):
      jt = pl.multiple_of((j >> 7) << 7, 128)
      pltpu.async_copy(tabt_hbm.at[:, pl.ds(jt, 128)], buf, sem)

    def extract(j, r, buf):
      c = jnp.broadcast_to(j & 127, (16,))
      rr = jnp.broadcast_to(jnp.int32(r), (16,))
      v0 = plsc.load_gather(buf, [row16, c])
      v1 = plsc.load_gather(buf, [row16 + 16, c])
      plsc.store_scatter(cols_v, [row16, rr], v0)
      plsc.store_scatter(cols_v, [row16 + 16, rr], v1)

    depth = len(bufs)
    for r in range(depth - 1):
      fetch(all_ids[r][0], bufs[r], sems[r])
    for r in range(_BPW):
      j, _ = all_ids[r]
      if r + depth - 1 < _BPW:
        fetch(all_ids[r + depth - 1][0], bufs[(r + depth - 1) % depth],
              sems[(r + depth - 1) % depth])
      pltpu.make_async_copy(tabt_hbm.at[:, pl.ds(0, 128)],
                            bufs[r % depth], sems[r % depth]).wait()
      extract(j, r, bufs[r % depth])
    pltpu.sync_copy(cols_v, out.at[:, pl.ds(base, _BPW)])

  tower(uid_hbm, utabt_hbm, uout)
  tower(iid_hbm, itabt_hbm, iout)


_sc_ids = functools.partial(
    pl.kernel,
    out_type=(
        jax.ShapeDtypeStruct((_D, _B), jnp.float32),
        jax.ShapeDtypeStruct((_D, _B), jnp.float32),
    ),
    mesh=plsc.VectorSubcoreMesh(core_axis_name="c", subcore_axis_name="s"),
    compiler_params=pltpu.CompilerParams(use_tc_tiling_on_sc=True,
                                         needs_layout_passes=False),
    scratch_types=[
        pltpu.VMEM((_BPW,), jnp.int32),
        pltpu.VMEM((_D, 128), jnp.float32),
        pltpu.VMEM((_D, 128), jnp.float32),
        pltpu.VMEM((_D, 128), jnp.float32),
        pltpu.VMEM((_D, 128), jnp.float32),
        pltpu.VMEM((_D, _BPW), jnp.float32),
        pltpu.SemaphoreType.DMA,
        pltpu.SemaphoreType.DMA,
        pltpu.SemaphoreType.DMA,
        pltpu.SemaphoreType.DMA,
    ],
)(_sc_id_body)


def _tc_body(uidt_ref, qmt_ref, iidt_ref, tmt_ref,
             uw1_ref, ub1_ref, uw2_ref, ub2_ref,
             iw1_ref, ib1_ref, iw2_ref, ib2_ref, out_ref):
  f32 = jnp.float32
  contract0 = (((0,), (0,)), ((), ()))

  def dnn_t(a, b, w1, b1, w2, b2):
    x = jnp.concatenate([a, b], axis=0)
    h = lax.dot_general(w1, x, contract0, preferred_element_type=f32)
    h = jnp.maximum(h + b1, 0.0)
    o = lax.dot_general(w2, h, contract0, preferred_element_type=f32)
    return jnp.maximum(o + b2, 0.0)

  uo = dnn_t(uidt_ref[...], qmt_ref[...], uw1_ref[...], ub1_ref[...],
             uw2_ref[...], ub2_ref[...])
  io = dnn_t(iidt_ref[...], tmt_ref[...], iw1_ref[...], ib1_ref[...],
             iw2_ref[...], ib2_ref[...])
  eps = jnp.float32(1e-12)
  q = io * lax.rsqrt(jnp.maximum(jnp.sum(io * io, axis=1, keepdims=True), eps))
  t = uo * lax.rsqrt(jnp.maximum(jnp.sum(uo * uo, axis=1, keepdims=True), eps))
  qn = q * lax.rsqrt(jnp.maximum(jnp.sum(q * q, axis=0, keepdims=True), eps))
  tn = t * lax.rsqrt(jnp.maximum(jnp.sum(t * t, axis=0, keepdims=True), eps))
  cos = -jnp.sum(qn * tn, axis=0, keepdims=True)
  out_ref[...] = jax.nn.sigmoid(cos).reshape(_B)


_tc_dense = pl.pallas_call(
    _tc_body,
    out_shape=jax.ShapeDtypeStruct((_B,), jnp.float32),
)


def kernel(user_id, query, item_id, title, text_embed, user_id_table,
           item_id_table, uW1, ub1, uW2, ub2, iW1, ib1, iW2, ib2):
  uidt, iidt = _sc_ids(user_id.reshape(-1), item_id.reshape(-1),
                       user_id_table.T, item_id_table.T)
  text_pad = jnp.pad(text_embed, ((0, 127 - (text_embed.shape[0] - 1) % 128),
                                  (0, 0)))
  qmt, tmt = _sc_text(query.T, title.T, text_pad)
  return _tc_dense(uidt, qmt, iidt, tmt,
                   uW1, ub1.reshape(-1, 1), uW2, ub2.reshape(-1, 1),
                   iW1, ib1.reshape(-1, 1), iW2, ib2.reshape(-1, 1))


# speedup vs baseline: 1.3042x; 1.3042x over previous
import functools

import jax
import jax.numpy as jnp
from jax import lax
from jax.experimental import pallas as pl
from jax.experimental.pallas import tpu as pltpu
from jax.experimental.pallas import tpu_sc as plsc

_B = 4096
_L = 50
_D = 32
_NC = 2
_NS = 16
_NW = _NC * _NS
_BPW = _B // _NW


def _sc_text_body(qt_hbm, tt_hbm, text_hbm, dep_hbm, qmean_out, tmean_out,
                  tok_idx, b0, b1, b2, b3, b4, b5, b6, b7,
                  mean_v, meant_v, s0, s1, s2, s3, s4, s5, s6, s7):
  del dep_hbm
  wid = lax.axis_index("s") * _NC + lax.axis_index("c")
  base = wid * _BPW
  inv_l = jnp.float32(1.0 / _L)
  zero = jnp.zeros((16,), jnp.float32)
  lane16 = lax.iota(jnp.int32, 16)
  lane32 = lane16 * _D
  bufs = (b0, b1, b2, b3, b4, b5, b6, b7)
  sems = (s0, s1, s2, s3, s4, s5, s6, s7)

  def tower(idxt_hbm, meanout):
    pltpu.sync_copy(idxt_hbm.at[:, pl.ds(base, _BPW)], tok_idx)

    def zinit(g, c):
      for u in range(8):
        mean_v[pl.ds(g * 128 + u * 16, 16)] = zero
      return c

    lax.fori_loop(0, 32, zinit, 0)

    def launch(k, t):
      pltpu.async_copy(text_hbm.at[tok_idx.at[t]], bufs[k], sems[k])

    def wait(k):
      pltpu.make_async_copy(text_hbm.at[tok_idx.at[0]], bufs[k],
                            sems[k]).wait()

    def acc(ks):
      def grp(g, c):
        pend = []
        for u in range(8):
          r = g * 4 + (u // 2)
          o = (u % 2) * 16
          v = bufs[ks[0]][r, pl.ds(o, 16)]
          for k in ks[1:]:
            v = v + bufs[k][r, pl.ds(o, 16)]
          pend.append((g * 128 + u * 16, v))
        for off, v in pend:
          plsc.addupdate(mean_v.at[pl.ds(off, 16)], v)
        return c

      lax.fori_loop(0, 32, grp, 0)

    for k in range(4):
      launch(k, k)

    def body(h, c):
      t0 = 8 * h
      for k in range(4):
        launch(4 + k, jnp.minimum(t0 + 4 + k, _L - 1))
      for k in range(4):
        wait(k)
      acc((0, 1, 2, 3))
      for k in range(4):
        launch(k, jnp.minimum(t0 + 8 + k, _L - 1))
      for k in range(4):
        wait(4 + k)
      acc((4, 5, 6, 7))
      return c

    lax.fori_loop(0, 6, body, 0)
    for k in range(4):
      wait(k)
    acc((0, 1))

    def tpose(f, c):
      fb = jnp.broadcast_to(f, (16,))
      for g in range(8):
        idx = lane32 + (g * 16 * _D + f)
        v = plsc.load_gather(mean_v, [idx])
        plsc.store_scatter(meant_v, [fb, lane16 + g * 16], v * inv_l)
      return c

    lax.fori_loop(0, _D, tpose, 0)
    pltpu.sync_copy(meant_v, meanout.at[:, pl.ds(base, _BPW)])

  tower(qt_hbm, qmean_out)
  tower(tt_hbm, tmean_out)


_sc_text = functools.partial(
    pl.kernel,
    out_type=(
        jax.ShapeDtypeStruct((_D, _B), jnp.float32),
        jax.ShapeDtypeStruct((_D, _B), jnp.float32),
    ),
    mesh=plsc.VectorSubcoreMesh(core_axis_name="c", subcore_axis_name="s"),
    compiler_params=pltpu.CompilerParams(use_tc_tiling_on_sc=False,
                                         needs_layout_passes=False),
    scratch_types=(
        [pltpu.VMEM((_L, _BPW), jnp.int32)]
        + [pltpu.VMEM((_BPW, _D), jnp.float32)] * 8
        + [pltpu.VMEM((_BPW * _D,), jnp.float32),
           pltpu.VMEM((_D, _BPW), jnp.float32)]
        + [pltpu.SemaphoreType.DMA] * 8
    ),
)(_sc_text_body)


def _sc_id_body(uid_hbm, iid_hbm, utabt_hbm, itabt_hbm,
                uout, iout, idx_v, tile_a, tile_b, tile_c, tile_d, cols_v,
                sem_a, sem_b, sem_c, sem_d):
  wid = lax.axis_index("s") * _NC + lax.axis_index("c")
  base = wid * _BPW
  row16 = lax.iota(jnp.int32, 16)

  def tower(id_hbm, tabt_hbm, out):
    pltpu.sync_copy(id_hbm.at[pl.ds(base, _BPW)], idx_v)
    bufs = (tile_a, tile_b, tile_c, tile_d)
    sems = (sem_a, sem_b, sem_c, sem_d)
    all_ids = []
    for g in range(_BPW // 16):
      ids = idx_v[pl.ds(g * 16, 16)]
      all_ids.extend((ids[k], g * 16 + k) for k in range(16))

    def fetch(j, buf, sem):
      jt = pl.multiple_of((j >> 7) << 7, 128)
      pltpu.async_copy(tabt_hbm.at[:, pl.ds(jt, 128)], buf, sem)

    def extract(j, r, buf):
      c = jnp.broadcast_to(j & 127, (16,))
      rr = jnp.broadcast_to(jnp.int32(r), (16,))
      v0 = plsc.load_gather(buf, [row16, c])
      v1 = plsc.load_gather(buf, [row16 + 16, c])
      plsc.store_scatter(cols_v, [row16, rr], v0)
      plsc.store_scatter(cols_v, [row16 + 16, rr], v1)

    depth = len(bufs)
    for r in range(depth - 1):
      fetch(all_ids[r][0], bufs[r], sems[r])
    for r in range(_BPW):
      j, _ = all_ids[r]
      if r + depth - 1 < _BPW:
        fetch(all_ids[r + depth - 1][0], bufs[(r + depth - 1) % depth],
              sems[(r + depth - 1) % depth])
      pltpu.make_async_copy(tabt_hbm.at[:, pl.ds(0, 128)],
                            bufs[r % depth], sems[r % depth]).wait()
      extract(j, r, bufs[r % depth])
    pltpu.sync_copy(cols_v, out.at[:, pl.ds(base, _BPW)])

  tower(uid_hbm, utabt_hbm, uout)
  tower(iid_hbm, itabt_hbm, iout)


_sc_ids = functools.partial(
    pl.kernel,
    out_type=(
        jax.ShapeDtypeStruct((_D, _B), jnp.float32),
        jax.ShapeDtypeStruct((_D, _B), jnp.float32),
    ),
    mesh=plsc.VectorSubcoreMesh(core_axis_name="c", subcore_axis_name="s"),
    compiler_params=pltpu.CompilerParams(use_tc_tiling_on_sc=True,
                                         needs_layout_passes=False),
    scratch_types=[
        pltpu.VMEM((_BPW,), jnp.int32),
        pltpu.VMEM((_D, 128), jnp.float32),
        pltpu.VMEM((_D, 128), jnp.float32),
        pltpu.VMEM((_D, 128), jnp.float32),
        pltpu.VMEM((_D, 128), jnp.float32),
        pltpu.VMEM((_D, _BPW), jnp.float32),
        pltpu.SemaphoreType.DMA,
        pltpu.SemaphoreType.DMA,
        pltpu.SemaphoreType.DMA,
        pltpu.SemaphoreType.DMA,
    ],
)(_sc_id_body)


def _tc_body(uidt_ref, qmt_ref, iidt_ref, tmt_ref,
             uw1_ref, ub1_ref, uw2_ref, ub2_ref,
             iw1_ref, ib1_ref, iw2_ref, ib2_ref, out_ref):
  f32 = jnp.float32
  contract0 = (((0,), (0,)), ((), ()))

  def dnn_t(a, b, w1, b1, w2, b2):
    x = jnp.concatenate([a, b], axis=0)
    h = lax.dot_general(w1, x, contract0, preferred_element_type=f32)
    h = jnp.maximum(h + b1, 0.0)
    o = lax.dot_general(w2, h, contract0, preferred_element_type=f32)
    return jnp.maximum(o + b2, 0.0)

  uo = dnn_t(uidt_ref[...], qmt_ref[...], uw1_ref[...], ub1_ref[...],
             uw2_ref[...], ub2_ref[...])
  io = dnn_t(iidt_ref[...], tmt_ref[...], iw1_ref[...], ib1_ref[...],
             iw2_ref[...], ib2_ref[...])
  eps = jnp.float32(1e-12)
  q = io * lax.rsqrt(jnp.maximum(jnp.sum(io * io, axis=1, keepdims=True), eps))
  t = uo * lax.rsqrt(jnp.maximum(jnp.sum(uo * uo, axis=1, keepdims=True), eps))
  qn = q * lax.rsqrt(jnp.maximum(jnp.sum(q * q, axis=0, keepdims=True), eps))
  tn = t * lax.rsqrt(jnp.maximum(jnp.sum(t * t, axis=0, keepdims=True), eps))
  cos = -jnp.sum(qn * tn, axis=0, keepdims=True)
  out_ref[...] = jax.nn.sigmoid(cos).reshape(_B)


_tc_dense = pl.pallas_call(
    _tc_body,
    out_shape=jax.ShapeDtypeStruct((_B,), jnp.float32),
)


def kernel(user_id, query, item_id, title, text_embed, user_id_table,
           item_id_table, uW1, ub1, uW2, ub2, iW1, ib1, iW2, ib2):
  uidt, iidt = _sc_ids(user_id.reshape(-1), item_id.reshape(-1),
                       user_id_table.T, item_id_table.T)
  qmt, tmt = _sc_text(query.T, title.T, text_embed, uidt)
  return _tc_dense(uidt, qmt, iidt, tmt,
                   uW1, ub1.reshape(-1, 1), uW2, ub2.reshape(-1, 1),
                   iW1, ib1.reshape(-1, 1), iW2, ib2.reshape(-1, 1))
